# baseline (speedup 1.0000x reference)
import functools

import jax
import jax.numpy as jnp
from jax import lax
from jax.experimental import pallas as pl
from jax.experimental.pallas import tpu as pltpu
from jax.experimental.pallas import tpu_sc as plsc

H = 1080
W = 1920
HW = H * W
N = 2_000_000
HWP = HW + N

NW = 32
PER_TILE = 62528
N_PAD = NW * PER_TILE
NCHUNKS = 4
CHUNK = PER_TILE // NCHUNKS

ROWS = N // 128
ROWS_PAD = N_PAD // 128
BLK = 512
GRID = -(-ROWS_PAD // BLK)


def _sc_gather_body(table, idx_hbm, out, idx_v, iidx_v, buf, sem):
    wid = lax.axis_index("s") * 2 + lax.axis_index("c")
    base = wid * PER_TILE
    lane3 = lax.iota(jnp.int32, 16) * 3

    def expand(it, _):
        v3 = idx_v[pl.ds(it * 16, 16)] * 3
        pos = lane3 + it * 48
        plsc.store_scatter(iidx_v, [pos], v3)
        plsc.store_scatter(iidx_v, [pos + 1], v3 + 1)
        plsc.store_scatter(iidx_v, [pos + 2], v3 + 2)
        return 0

    for j in range(NCHUNKS):
        off = base + j * CHUNK
        pltpu.sync_copy(idx_hbm.at[pl.ds(off, CHUNK)], idx_v)
        lax.fori_loop(0, CHUNK // 16, expand, 0)
        pltpu.async_copy(table.at[iidx_v], buf, sem).wait()
        pltpu.sync_copy(buf, out.at[pl.ds(3 * off, 3 * CHUNK)])


_sc_gather = functools.partial(
    pl.kernel,
    mesh=plsc.VectorSubcoreMesh(core_axis_name="c", subcore_axis_name="s"),
    compiler_params=pltpu.CompilerParams(needs_layout_passes=False),
    out_type=jax.ShapeDtypeStruct((3 * N_PAD,), jnp.float32),
    scratch_types=[
        pltpu.VMEM((CHUNK,), jnp.int32),
        pltpu.VMEM((3 * CHUNK,), jnp.int32),
        pltpu.VMEM((3 * CHUNK,), jnp.float32),
        pltpu.SemaphoreType.DMA,
    ],
)(_sc_gather_body)


def _ew_body(c_ref, s_ref, m_ref, t_ref, dmin_ref, pix_ref,
             nmax_ref, ntot_ref, nmin_ref, idx2_ref):
    c = c_ref[...]
    m = m_ref[...]
    mask = c > m
    nmax_ref[...] = jnp.where(mask, c, m)
    ntot_ref[...] = t_ref[...] + c
    s = s_ref[...]
    d = dmin_ref[...]
    nmin_ref[...] = jnp.where(s < d, s, d)
    i = pl.program_id(0)
    row = lax.broadcasted_iota(jnp.int32, (BLK, 128), 0)
    lane = lax.broadcasted_iota(jnp.int32, (BLK, 128), 1)
    gidx = (i * BLK + row) * 128 + lane
    idx2 = jnp.where(mask, pix_ref[...], HW + gidx)
    idx2_ref[...] = jnp.clip(idx2, 0, HWP - 1)


def _ew_call(c, s, m, t, dmin, pix):
    flat_spec = pl.BlockSpec((BLK, 128), lambda i: (i, 0))
    return pl.pallas_call(
        _ew_body,
        grid=(GRID,),
        in_specs=[flat_spec] * 6,
        out_specs=[flat_spec] * 4,
        out_shape=[
            jax.ShapeDtypeStruct((ROWS, 128), jnp.float32),
            jax.ShapeDtypeStruct((ROWS, 128), jnp.float32),
            jax.ShapeDtypeStruct((ROWS, 128), jnp.float32),
            jax.ShapeDtypeStruct((ROWS_PAD, 128), jnp.int32),
        ],
    )(c, s, m, t, dmin, pix)


def kernel(colour, current_gauss_contributions, current_gauss_surface_distances,
           gaussian_max_contribution, gaussian_colours, gaussian_total_contribution,
           gaussian_min_surface_distance, current_gauss_pixels):
    colour_flat = jnp.transpose(colour.reshape(3, HW))
    table = jnp.concatenate(
        [colour_flat, gaussian_colours], axis=0).reshape(3 * HWP)

    r = lambda x: x.reshape(ROWS, 128)
    nmax, ntot, nmin, idx2 = _ew_call(
        r(current_gauss_contributions),
        r(current_gauss_surface_distances),
        r(gaussian_max_contribution),
        r(gaussian_total_contribution),
        r(gaussian_min_surface_distance),
        r(current_gauss_pixels))

    gathered = _sc_gather(table, idx2.reshape(N_PAD))
    new_colours = gathered[:3 * N].reshape(N, 3)
    return (nmax.reshape(N), new_colours, ntot.reshape(N), nmin.reshape(N))


# speedup vs baseline: 2.3491x; 2.3491x over previous
import functools

import jax
import jax.numpy as jnp
from jax import lax
from jax.experimental import pallas as pl
from jax.experimental.pallas import tpu as pltpu
from jax.experimental.pallas import tpu_sc as plsc

H = 1080
W = 1920
HW = H * W
N = 2_000_000

NW = 32
G_MAIN = 64_000
G_TAIL = N - 31 * G_MAIN
CHUNK = 8_000
NCH_MAIN = G_MAIN // CHUNK
NCH_TAIL = G_TAIL // CHUNK

ROWS = N // 128
BLK = 512
GRID = -(-ROWS // BLK)


def _sc_body(p0, p1, p2, t0, t1, t2, gidx_hbm, mflag_hbm, out,
             gv, mv, tb0, tb1, tb2, gb0, gb1, gb2, ob,
             s0, s1, s2):
    wid = lax.axis_index("s") * 2 + lax.axis_index("c")
    base = wid * G_MAIN
    lane3 = lax.iota(jnp.int32, 16) * 3

    def select_interleave(it, _):
        mvv = mv[pl.ds(it * 16, 16)] != 0
        pos = lane3 + it * 48
        v0 = jnp.where(mvv, gb0[pl.ds(it * 16, 16)], tb0[pl.ds(it * 16, 16)])
        plsc.store_scatter(ob, [pos], v0)
        v1 = jnp.where(mvv, gb1[pl.ds(it * 16, 16)], tb1[pl.ds(it * 16, 16)])
        plsc.store_scatter(ob, [pos + 1], v1)
        v2 = jnp.where(mvv, gb2[pl.ds(it * 16, 16)], tb2[pl.ds(it * 16, 16)])
        plsc.store_scatter(ob, [pos + 2], v2)
        return 0

    for j in range(NCH_MAIN):
        @pl.when(jnp.logical_or(wid < 31, j < NCH_TAIL))
        def _():
            off = base + j * CHUNK
            pltpu.sync_copy(gidx_hbm.at[pl.ds(off, CHUNK)], gv)
            c0 = pltpu.async_copy(p0.at[gv], gb0, s0)
            c1 = pltpu.async_copy(p1.at[gv], gb1, s1)
            c2 = pltpu.async_copy(p2.at[gv], gb2, s2)
            pltpu.sync_copy(mflag_hbm.at[pl.ds(off, CHUNK)], mv)
            pltpu.sync_copy(t0.at[pl.ds(off, CHUNK)], tb0)
            pltpu.sync_copy(t1.at[pl.ds(off, CHUNK)], tb1)
            pltpu.sync_copy(t2.at[pl.ds(off, CHUNK)], tb2)
            c0.wait()
            c1.wait()
            c2.wait()
            lax.fori_loop(0, CHUNK // 16, select_interleave, 0)
            pltpu.sync_copy(ob, out.at[pl.ds(3 * off, 3 * CHUNK)])


_sc_call = functools.partial(
    pl.kernel,
    mesh=plsc.VectorSubcoreMesh(core_axis_name="c", subcore_axis_name="s"),
    compiler_params=pltpu.CompilerParams(needs_layout_passes=False),
    out_type=jax.ShapeDtypeStruct((3 * N,), jnp.float32),
    scratch_types=[
        pltpu.VMEM((CHUNK,), jnp.int32),
        pltpu.VMEM((CHUNK,), jnp.int32),
        pltpu.VMEM((CHUNK,), jnp.float32),
        pltpu.VMEM((CHUNK,), jnp.float32),
        pltpu.VMEM((CHUNK,), jnp.float32),
        pltpu.VMEM((CHUNK,), jnp.float32),
        pltpu.VMEM((CHUNK,), jnp.float32),
        pltpu.VMEM((CHUNK,), jnp.float32),
        pltpu.VMEM((3 * CHUNK,), jnp.float32),
        pltpu.SemaphoreType.DMA,
        pltpu.SemaphoreType.DMA,
        pltpu.SemaphoreType.DMA,
    ],
)(_sc_body)


def _ew_body(c_ref, s_ref, m_ref, t_ref, dmin_ref, pix_ref,
             nmax_ref, ntot_ref, nmin_ref, gidx_ref, mflag_ref):
    c = c_ref[...]
    m = m_ref[...]
    mask = c > m
    nmax_ref[...] = jnp.where(mask, c, m)
    ntot_ref[...] = t_ref[...] + c
    s = s_ref[...]
    d = dmin_ref[...]
    nmin_ref[...] = jnp.where(s < d, s, d)
    pix = pix_ref[...]
    gidx_ref[...] = jnp.where(mask, jnp.clip(pix, 0, HW - 1), 0)
    mflag_ref[...] = mask.astype(jnp.int32)


def _ew_call(c, s, m, t, dmin, pix):
    flat_spec = pl.BlockSpec((BLK, 128), lambda i: (i, 0))
    f32_out = jax.ShapeDtypeStruct((ROWS, 128), jnp.float32)
    i32_out = jax.ShapeDtypeStruct((ROWS, 128), jnp.int32)
    return pl.pallas_call(
        _ew_body,
        grid=(GRID,),
        in_specs=[flat_spec] * 6,
        out_specs=[flat_spec] * 5,
        out_shape=[f32_out, f32_out, f32_out, i32_out, i32_out],
    )(c, s, m, t, dmin, pix)


def kernel(colour, current_gauss_contributions, current_gauss_surface_distances,
           gaussian_max_contribution, gaussian_colours, gaussian_total_contribution,
           gaussian_min_surface_distance, current_gauss_pixels):
    planes = colour.reshape(3, HW)
    t_old = gaussian_colours.T

    r = lambda x: x.reshape(ROWS, 128)
    nmax, ntot, nmin, gidx, mflag = _ew_call(
        r(current_gauss_contributions),
        r(current_gauss_surface_distances),
        r(gaussian_max_contribution),
        r(gaussian_total_contribution),
        r(gaussian_min_surface_distance),
        r(current_gauss_pixels))

    out_flat = _sc_call(planes[0], planes[1], planes[2],
                        t_old[0], t_old[1], t_old[2],
                        gidx.reshape(N), mflag.reshape(N))
    return (nmax.reshape(N), out_flat.reshape(N, 3), ntot.reshape(N),
            nmin.reshape(N))


# speedup vs baseline: 2.8931x; 1.2316x over previous
import functools

import jax
import jax.numpy as jnp
from jax import lax
from jax.experimental import pallas as pl
from jax.experimental.pallas import tpu as pltpu
from jax.experimental.pallas import tpu_sc as plsc

H = 1080
W = 1920
HW = H * W
N = 2_000_000

NW = 32
PER_TILE = 62528
N_PAD = NW * PER_TILE
CHUNK = 7816
NCHUNKS = PER_TILE // CHUNK

ROWS = N // 128
ROWS_PAD = N_PAD // 128
BLK = 512
GRID = -(-ROWS // BLK)


def _pre_body(c_ref, m_ref, pix_ref, gidx_ref):
    mask = c_ref[...] > m_ref[...]
    gidx_ref[...] = jnp.where(mask, jnp.clip(pix_ref[...], 0, HW - 1), 0)


def _pre_call(c, m, pix):
    flat_spec = pl.BlockSpec((BLK, 128), lambda i: (i, 0))
    return pl.pallas_call(
        _pre_body,
        grid=(GRID,),
        in_specs=[flat_spec] * 3,
        out_specs=flat_spec,
        out_shape=jax.ShapeDtypeStruct((ROWS_PAD, 128), jnp.int32),
    )(c, m, pix)


def _sc_gather_body(p0, p1, p2, idx_hbm, g0, g1, g2, idx_v, b0, b1, b2,
                    s0, s1, s2):
    wid = lax.axis_index("s") * 2 + lax.axis_index("c")
    base = wid * PER_TILE
    for j in range(NCHUNKS):
        off = base + j * CHUNK
        pltpu.sync_copy(idx_hbm.at[pl.ds(off, CHUNK)], idx_v)
        cp0 = pltpu.async_copy(p0.at[idx_v], b0, s0)
        cp1 = pltpu.async_copy(p1.at[idx_v], b1, s1)
        cp2 = pltpu.async_copy(p2.at[idx_v], b2, s2)
        cp0.wait()
        cp1.wait()
        cp2.wait()
        pltpu.sync_copy(b0, g0.at[pl.ds(off, CHUNK)])
        pltpu.sync_copy(b1, g1.at[pl.ds(off, CHUNK)])
        pltpu.sync_copy(b2, g2.at[pl.ds(off, CHUNK)])


_sc_gather = functools.partial(
    pl.kernel,
    mesh=plsc.VectorSubcoreMesh(core_axis_name="c", subcore_axis_name="s"),
    out_type=[jax.ShapeDtypeStruct((N_PAD,), jnp.float32)] * 3,
    scratch_types=[
        pltpu.VMEM((CHUNK,), jnp.int32),
        pltpu.VMEM((CHUNK,), jnp.float32),
        pltpu.VMEM((CHUNK,), jnp.float32),
        pltpu.VMEM((CHUNK,), jnp.float32),
        pltpu.SemaphoreType.DMA,
        pltpu.SemaphoreType.DMA,
        pltpu.SemaphoreType.DMA,
    ],
)(_sc_gather_body)


def _ew_body(c_ref, s_ref, m_ref, t_ref, dmin_ref, g0_ref, g1_ref, g2_ref,
             oldt_ref, nmax_ref, ntot_ref, nmin_ref, ncolt_ref):
    c = c_ref[...]
    m = m_ref[...]
    mask = c > m
    nmax_ref[...] = jnp.where(mask, c, m)
    ntot_ref[...] = t_ref[...] + c
    s = s_ref[...]
    d = dmin_ref[...]
    nmin_ref[...] = jnp.where(s < d, s, d)
    ncolt_ref[0] = jnp.where(mask, g0_ref[...], oldt_ref[0])
    ncolt_ref[1] = jnp.where(mask, g1_ref[...], oldt_ref[1])
    ncolt_ref[2] = jnp.where(mask, g2_ref[...], oldt_ref[2])


def _ew_call(c, s, m, t, dmin, g0, g1, g2, oldt):
    flat_spec = pl.BlockSpec((BLK, 128), lambda i: (i, 0))
    col_spec = pl.BlockSpec((3, BLK, 128), lambda i: (0, i, 0))
    return pl.pallas_call(
        _ew_body,
        grid=(GRID,),
        in_specs=[flat_spec] * 8 + [col_spec],
        out_specs=[flat_spec] * 3 + [col_spec],
        out_shape=[
            jax.ShapeDtypeStruct((ROWS, 128), jnp.float32),
            jax.ShapeDtypeStruct((ROWS, 128), jnp.float32),
            jax.ShapeDtypeStruct((ROWS, 128), jnp.float32),
            jax.ShapeDtypeStruct((3, ROWS, 128), jnp.float32),
        ],
    )(c, s, m, t, dmin, g0, g1, g2, oldt)


def kernel(colour, current_gauss_contributions, current_gauss_surface_distances,
           gaussian_max_contribution, gaussian_colours, gaussian_total_contribution,
           gaussian_min_surface_distance, current_gauss_pixels):
    planes = colour.reshape(3, HW)
    r0 = lambda x: x.reshape(ROWS, 128)
    gidx = _pre_call(r0(current_gauss_contributions),
                     r0(gaussian_max_contribution),
                     r0(current_gauss_pixels))
    g0, g1, g2 = _sc_gather(planes[0], planes[1], planes[2],
                            gidx.reshape(N_PAD))

    r = lambda x: x.reshape(ROWS, 128)
    rp = lambda x: x.reshape(N_PAD // 128, 128)
    oldt = gaussian_colours.T.reshape(3, ROWS, 128)
    nmax, ntot, nmin, ncolt = _ew_call(
        r(current_gauss_contributions),
        r(current_gauss_surface_distances),
        r(gaussian_max_contribution),
        r(gaussian_total_contribution),
        r(gaussian_min_surface_distance),
        rp(g0), rp(g1), rp(g2), oldt)

    new_colours = ncolt.reshape(3, N).T
    return (nmax.reshape(N), new_colours, ntot.reshape(N), nmin.reshape(N))
